# baseline (speedup 1.0000x reference)
import functools

import jax
import jax.numpy as jnp
from jax import lax
from jax.experimental import pallas as pl
from jax.experimental.pallas import tpu as pltpu
from jax.experimental.pallas import tpu_sc as plsc

VOCAB_SIZE = 64
HIDDEN_DIM = 64

_B = 16384
_T = 200
_N = _B * _T

_SC_XROWS = 5632
_K = 4
_NW = 32
_ROWS_PER_W = _SC_XROWS // _NW
_OUTER = _ROWS_PER_W // _K
_CHUNK = _K * _T


def _sc_body(x_hbm, table_hbm, out_hbm, idx_v, rows_v, table_s,
             sem_idx, sem_g, sem_out):
    wid = lax.axis_index("s") * 2 + lax.axis_index("c")
    w_base = wid * _ROWS_PER_W

    @pl.when(lax.axis_index("s") == 0)
    def _():
        pltpu.sync_copy(table_hbm, table_s)
    plsc.subcore_barrier()

    def idx_copy(slot, row):
        return pltpu.make_async_copy(
            x_hbm.at[pl.ds(row, _K)], idx_v.at[slot], sem_idx)

    def out_copy(slot, row):
        return pltpu.make_async_copy(
            rows_v.at[slot], out_hbm.at[pl.ds(row * _T, _CHUNK)], sem_out)

    idx_copy(0, w_base).start()
    idx_copy(1, w_base + _K).start()

    def body(i, carry):
        for u in range(4):
            it = i * 4 + u
            rb = u % 2
            sb = u
            row = w_base + it * _K
            idx_copy(sb, row).wait()

            @pl.when(it >= 2)
            def _():
                out_copy(rb, row - 2 * _K).wait()

            gathers = []
            for j in range(_K):
                for off, width in ((0, 128), (128, 72)):
                    gathers.append(pltpu.async_copy(
                        table_s.at[idx_v.at[sb, j, pl.ds(off, width)]],
                        rows_v.at[rb, pl.ds(j * _T + off, width)],
                        sem_g))

            @pl.when(it + 2 < _OUTER)
            def _():
                idx_copy((u + 2) % 4, row + 2 * _K).start()

            for g in gathers:
                g.wait()
            out_copy(rb, row).start()
        return carry

    lax.fori_loop(0, _OUTER // 4, body, 0)
    out_copy(0, w_base + (_OUTER - 2) * _K).wait()
    out_copy(1, w_base + (_OUTER - 1) * _K).wait()


def _run_sc(x, table):
    mesh = plsc.VectorSubcoreMesh(core_axis_name="c", subcore_axis_name="s")
    return functools.partial(
        pl.kernel,
        mesh=mesh,
        out_type=jax.ShapeDtypeStruct((_N, HIDDEN_DIM), jnp.float32),
        scratch_types=[
            pltpu.VMEM((4, _K, _T), jnp.int32),
            pltpu.VMEM((2, _CHUNK, HIDDEN_DIM), jnp.float32),
            pltpu.VMEM_SHARED((VOCAB_SIZE, HIDDEN_DIM), jnp.float32),
            pltpu.SemaphoreType.DMA,
            pltpu.SemaphoreType.DMA,
            pltpu.SemaphoreType.DMA,
        ],
        compiler_params=pltpu.CompilerParams(use_tc_tiling_on_sc=False),
    )(_sc_body)(x, table)


_SR = 32
_TC_OFF = _SC_XROWS // _SR


def _tc_body(x_ref, hi_ref, buf_ref, o_ref):
    dn = (((0,), (0,)), ((), ()))
    iota = jax.lax.broadcasted_iota(jnp.int32, (VOCAB_SIZE, _T), 0)
    for j in range(_SR):
        oh = (x_ref[j, :][None, :] == iota).astype(jnp.bfloat16)
        acc = jax.lax.dot_general(oh, hi_ref[...], dn,
                                  preferred_element_type=jnp.float32)
        o_ref[pl.ds(j * _T, _T), :] = acc


def _run_tc(x, table, buf):
    hi = table.astype(jnp.bfloat16)
    return pl.pallas_call(
        _tc_body,
        grid=((_B - _SC_XROWS) // _SR,),
        in_specs=[
            pl.BlockSpec((_SR, _T), lambda i: (i + _TC_OFF, 0)),
            pl.BlockSpec((VOCAB_SIZE, HIDDEN_DIM), lambda i: (0, 0)),
            pl.BlockSpec(memory_space=pltpu.MemorySpace.HBM),
        ],
        out_specs=pl.BlockSpec((_SR * _T, HIDDEN_DIM),
                               lambda i: (i + _TC_OFF, 0)),
        out_shape=jax.ShapeDtypeStruct((_N, HIDDEN_DIM), jnp.float32),
        input_output_aliases={2: 0},
    )(x, hi, buf)


def kernel(x, table):
    buf = _run_sc(x, table)
    out = _run_tc(x, table, buf)
    return out.reshape(_B, _T, HIDDEN_DIM)


# speedup vs baseline: 1.1074x; 1.1074x over previous
import functools

import jax
import jax.numpy as jnp
from jax import lax
from jax.experimental import pallas as pl
from jax.experimental.pallas import tpu as pltpu
from jax.experimental.pallas import tpu_sc as plsc

VOCAB_SIZE = 64
HIDDEN_DIM = 64

_LANE = 128
_K = 4
_TOKENS = 16384 * 200
_ROWS = _TOKENS // _LANE
_NW = 32
_ROWS_PER_W = _ROWS // _NW
_OUTER = _ROWS_PER_W // _K


def _emb_body(x_hbm, table_hbm, out_hbm, idx_v, rows_v, table_s,
              sem_idx, sem_g, sem_out):
    wid = lax.axis_index("s") * 2 + lax.axis_index("c")
    w_base = wid * _ROWS_PER_W

    @pl.when(lax.axis_index("s") == 0)
    def _():
        pltpu.sync_copy(table_hbm, table_s)
    plsc.subcore_barrier()

    def idx_copy(slot, base):
        return pltpu.make_async_copy(
            x_hbm.at[pl.ds(base, _K)], idx_v.at[slot], sem_idx)

    def out_copy(slot, base):
        return pltpu.make_async_copy(
            rows_v.at[slot], out_hbm.at[pl.ds(base, _K)], sem_out)

    idx_copy(0, w_base).start()
    idx_copy(1, w_base + _K).start()

    def body(i, carry):
        for u in range(4):
            it = i * 4 + u
            rb = u % 2
            sb = u
            base = w_base + it * _K
            idx_copy(sb, base).wait()

            @pl.when(it >= 2)
            def _():
                out_copy(rb, base - 2 * _K).wait()

            gathers = [
                pltpu.async_copy(
                    table_s.at[idx_v.at[sb, j]], rows_v.at[rb, j], sem_g)
                for j in range(_K)
            ]

            @pl.when(it + 2 < _OUTER)
            def _():
                idx_copy((u + 2) % 4, base + 2 * _K).start()

            for g in gathers:
                g.wait()
            out_copy(rb, base).start()
        return carry

    lax.fori_loop(0, _OUTER // 4, body, 0)
    out_copy(0, w_base + (_OUTER - 2) * _K).wait()
    out_copy(1, w_base + (_OUTER - 1) * _K).wait()


def kernel(x, table):
    x2 = x.reshape(_ROWS, _LANE).astype(jnp.int32)
    mesh = plsc.VectorSubcoreMesh(core_axis_name="c", subcore_axis_name="s")
    run = functools.partial(
        pl.kernel,
        mesh=mesh,
        out_type=jax.ShapeDtypeStruct((_ROWS, _LANE, HIDDEN_DIM), jnp.float32),
        scratch_types=[
            pltpu.VMEM((4, _K, _LANE), jnp.int32),
            pltpu.VMEM((2, _K, _LANE, HIDDEN_DIM), jnp.float32),
            pltpu.VMEM_SHARED((VOCAB_SIZE, HIDDEN_DIM), jnp.float32),
            pltpu.SemaphoreType.DMA,
            pltpu.SemaphoreType.DMA,
            pltpu.SemaphoreType.DMA,
        ],
        compiler_params=pltpu.CompilerParams(use_tc_tiling_on_sc=False),
    )(_emb_body)
    out = run(x2, table)
    return out.reshape(16384, 200, HIDDEN_DIM)
